# Initial kernel scaffold; baseline (speedup 1.0000x reference)
import functools

import jax
import jax.numpy as jnp
from jax import lax
from jax.experimental import pallas as pl
from jax.experimental.pallas import tpu as pltpu
from jax.experimental.pallas import tpu_sc as plsc

N = 10000
E = 320000
NG = 16
NHID = 128
NMACRO = 512

NC = 2
NS = 16
NW = NC * NS
PER_W = E // NW
CH = 80
NCHUNK = PER_W // CH
ROWS_W = N // NS

BN = 1000
BE = 2000

_F32 = jnp.float32


def _node_body(xf_ref, macro_ref, W_ref, b_ref, att1_ref,
               h_ref, ha_ref, cnt_ref):
    i = pl.program_id(0)
    rid = lax.broadcasted_iota(jnp.int32, (BN, NMACRO), 0) + i * BN
    eqf = (rid == macro_ref[...]).astype(_F32)
    ones_m = jnp.ones((1, NMACRO), _F32)
    dn = (((1,), (1,)), ((), ()))
    cnt = lax.dot_general(eqf, ones_m, dn, preferred_element_type=_F32)
    ismacro = (cnt > 0).astype(_F32)
    h = jnp.dot(xf_ref[...], W_ref[...], preferred_element_type=_F32)
    h = h + ismacro * W_ref[NHID - 1:NHID, :] + b_ref[...]
    h_ref[...] = h
    ha_ref[...] = lax.dot_general(h, att1_ref[...], dn,
                                  preferred_element_type=_F32)
    cnt_ref[...] = cnt


def _node_call(xf, macro2, W, b2, att1):
    return pl.pallas_call(
        _node_body,
        grid=(N // BN,),
        in_specs=[
            pl.BlockSpec((BN, NHID), lambda i: (i, 0)),
            pl.BlockSpec((1, NMACRO), lambda i: (0, 0)),
            pl.BlockSpec((NHID, NHID), lambda i: (0, 0)),
            pl.BlockSpec((1, NHID), lambda i: (0, 0)),
            pl.BlockSpec((1, NHID), lambda i: (0, 0)),
        ],
        out_specs=[
            pl.BlockSpec((BN, NHID), lambda i: (i, 0)),
            pl.BlockSpec((BN, 1), lambda i: (i, 0)),
            pl.BlockSpec((BN, 1), lambda i: (i, 0)),
        ],
        out_shape=[
            jax.ShapeDtypeStruct((N, NHID), _F32),
            jax.ShapeDtypeStruct((N, 1), _F32),
            jax.ShapeDtypeStruct((N, 1), _F32),
        ],
    )(xf, macro2, W, b2, att1)


_MESH = plsc.VectorSubcoreMesh(core_axis_name="c", subcore_axis_name="s",
                               num_cores=NC, num_subcores=NS)


@functools.partial(
    pl.kernel,
    out_type=(jax.ShapeDtypeStruct((E, NHID), _F32),
              jax.ShapeDtypeStruct((E,), _F32)),
    mesh=_MESH,
    scratch_types=[
        pltpu.VMEM((NCHUNK, CH), jnp.int32),
        pltpu.VMEM((PER_W,), jnp.int32),
        pltpu.VMEM((N,), _F32),
        pltpu.VMEM((PER_W,), _F32),
        pltpu.VMEM((CH, NHID), _F32),
        pltpu.SemaphoreType.DMA,
    ],
)
def _gather_kernel(h_hbm, ha_hbm, src2_hbm, dst_hbm, hsrc_hbm, had_hbm,
                   sidx_v, didx_v, ha_v, had_v, rbuf, sem):
    wid = lax.axis_index("s") * NC + lax.axis_index("c")
    base = wid * PER_W
    pltpu.sync_copy(src2_hbm.at[pl.ds(wid * NCHUNK, NCHUNK)], sidx_v)
    pltpu.sync_copy(dst_hbm.at[pl.ds(base, PER_W)], didx_v)
    pltpu.sync_copy(ha_hbm, ha_v)

    def gath16(k, carry):
        idx16 = didx_v[pl.ds(k * 16, 16)]
        had_v[pl.ds(k * 16, 16)] = plsc.load_gather(ha_v, [idx16])
        return carry
    lax.fori_loop(0, PER_W // 16, gath16, 0)
    pltpu.sync_copy(had_v, had_hbm.at[pl.ds(base, PER_W)])

    def chunk(j, carry):
        pltpu.async_copy(h_hbm.at[sidx_v.at[j]], rbuf, sem).wait()
        pltpu.sync_copy(rbuf, hsrc_hbm.at[pl.ds(base + j * CH, CH)])
        return carry
    lax.fori_loop(0, NCHUNK, chunk, 0)


def _edge_body(hsrc_ref, pin_ref, had_ref, pW_ref, pb_ref, att2_ref,
               W1_ref, b1_ref, W2_ref, b2_ref, W3_ref, b3_ref,
               e_ref, wattr_ref, exv_ref):
    dn = (((1,), (1,)), ((), ()))
    pinp = jnp.dot(pin_ref[...], pW_ref[...],
                   preferred_element_type=_F32) + pb_ref[...]
    ea = jnp.maximum(hsrc_ref[...] + pinp, 0.0)
    z = had_ref[...] + lax.dot_general(ea, att2_ref[...], dn,
                                       preferred_element_type=_F32)
    z = jnp.where(z >= 0, z, 0.1 * z)
    ex = jnp.exp(z)
    wattr_ref[...] = ex * ea
    lane = lax.broadcasted_iota(jnp.int32, (BE, 16), 1)
    exv_ref[...] = jnp.where(lane == 0, ex, 0.0)
    v = jnp.dot(ea, W1_ref[...], preferred_element_type=_F32) + b1_ref[...]
    v = jnp.where(v >= 0, v, 0.1 * v)
    v = jnp.dot(v, W2_ref[...], preferred_element_type=_F32) + b2_ref[...]
    v = jnp.where(v >= 0, v, 0.1 * v)
    e_ref[...] = jnp.dot(v, W3_ref[...], preferred_element_type=_F32) + b3_ref[...]


def _edge_call(hsrc, pin, had2, pW, pb2, att2, W1, b12, W2, b22, W3, b32):
    full = lambda shape: pl.BlockSpec(shape, lambda i: (0, 0))
    return pl.pallas_call(
        _edge_body,
        grid=(E // BE,),
        in_specs=[
            pl.BlockSpec((BE, NHID), lambda i: (i, 0)),
            pl.BlockSpec((BE, 4), lambda i: (i, 0)),
            pl.BlockSpec((BE, 1), lambda i: (i, 0)),
            full((4, NHID)), full((1, NHID)), full((1, NHID)),
            full((NHID, 64)), full((1, 64)),
            full((64, 32)), full((1, 32)),
            full((32, 4)), full((1, 4)),
        ],
        out_specs=[
            pl.BlockSpec((BE, 4), lambda i: (i, 0)),
            pl.BlockSpec((BE, NHID), lambda i: (i, 0)),
            pl.BlockSpec((BE, 16), lambda i: (i, 0)),
        ],
        out_shape=[
            jax.ShapeDtypeStruct((E, 4), _F32),
            jax.ShapeDtypeStruct((E, NHID), _F32),
            jax.ShapeDtypeStruct((E, 16), _F32),
        ],
    )(hsrc, pin, had2, pW, pb2, att2, W1, b12, W2, b22, W3, b32)


@functools.partial(
    pl.kernel,
    out_type=(jax.ShapeDtypeStruct((NC * N, NHID), _F32),
              jax.ShapeDtypeStruct((NC * N, 16), _F32)),
    mesh=_MESH,
    scratch_types=[
        pltpu.VMEM_SHARED((N, NHID), _F32),
        pltpu.VMEM_SHARED((N, 16), _F32),
        pltpu.VMEM((NCHUNK, CH), jnp.int32),
        pltpu.VMEM((CH, NHID), _F32),
        pltpu.VMEM((CH, 16), _F32),
    ],
)
def _scatter_kernel(wattr_hbm, exv_hbm, dst2_hbm, z128_hbm, z16_hbm,
                    sparts_hbm, dparts_hbm,
                    s_sh, t_sh, didx_v, rbuf, ebuf):
    core = lax.axis_index("c")
    sub = lax.axis_index("s")
    wid = sub * NC + core
    stripe = pl.ds(sub * ROWS_W, ROWS_W)
    pltpu.sync_copy(z128_hbm, s_sh.at[stripe])
    pltpu.sync_copy(z16_hbm, t_sh.at[stripe])
    plsc.subcore_barrier()

    base = wid * PER_W
    pltpu.sync_copy(dst2_hbm.at[pl.ds(wid * NCHUNK, NCHUNK)], didx_v)

    def chunk(j, carry):
        pltpu.sync_copy(wattr_hbm.at[pl.ds(base + j * CH, CH)], rbuf)
        pltpu.sync_copy(exv_hbm.at[pl.ds(base + j * CH, CH)], ebuf)
        pltpu.sync_copy(rbuf, s_sh.at[didx_v.at[j]], add=True)
        pltpu.sync_copy(ebuf, t_sh.at[didx_v.at[j]], add=True)
        return carry
    lax.fori_loop(0, NCHUNK, chunk, 0)
    plsc.subcore_barrier()

    out_rows = pl.ds(core * N + sub * ROWS_W, ROWS_W)
    pltpu.sync_copy(s_sh.at[stripe], sparts_hbm.at[out_rows])
    pltpu.sync_copy(t_sh.at[stripe], dparts_hbm.at[out_rows])


def _head_body(s0_ref, s1_ref, d0_ref, d1_ref, cnt_ref, batch_ref,
               eW_ref, eb_ref, W1_ref, b1_ref, W2_ref, b2_ref, W3_ref, b3_ref,
               g_ref, pm_acc, pf_acc, cm_acc, cf_acc):
    i = pl.program_id(0)

    @pl.when(i == 0)
    def _init():
        pm_acc[...] = jnp.zeros_like(pm_acc)
        pf_acc[...] = jnp.zeros_like(pf_acc)
        cm_acc[...] = jnp.zeros_like(cm_acc)
        cf_acc[...] = jnp.zeros_like(cf_acc)

    s = s0_ref[...] + s1_ref[...]
    t = jnp.sum(d0_ref[...] + d1_ref[...], axis=1, keepdims=True)
    xo = (jnp.dot(s, eW_ref[...], preferred_element_type=_F32)
          + t * eb_ref[...]) / (t + 1e-16)
    gi = lax.broadcasted_iota(jnp.int32, (BN, NG), 1)
    oh = (batch_ref[...] == gi).astype(_F32)
    ohm = oh * cnt_ref[...]
    ones = jnp.ones((BN, NHID), _F32)
    dn = (((0,), (0,)), ((), ()))
    pf_acc[...] += lax.dot_general(oh, xo, dn, preferred_element_type=_F32)
    pm_acc[...] += lax.dot_general(ohm, xo, dn, preferred_element_type=_F32)
    cf_acc[...] += lax.dot_general(oh, ones, dn, preferred_element_type=_F32)
    cm_acc[...] += lax.dot_general(ohm, ones, dn, preferred_element_type=_F32)

    @pl.when(i == N // BN - 1)
    def _final():
        x1a = pm_acc[...] / jnp.maximum(cm_acc[...], 1.0)
        x1b = pf_acc[...] / jnp.maximum(cf_acc[...], 1.0)
        v = (jnp.dot(x1a, W1_ref[...][:NHID], preferred_element_type=_F32)
             + jnp.dot(x1b, W1_ref[...][NHID:], preferred_element_type=_F32)
             + b1_ref[...])
        v = jnp.where(v >= 0, v, 0.1 * v)
        v = jnp.dot(v, W2_ref[...], preferred_element_type=_F32) + b2_ref[...]
        v = jnp.where(v >= 0, v, 0.1 * v)
        g_ref[...] = jnp.dot(v, W3_ref[...],
                             preferred_element_type=_F32) + b3_ref[...]


def _head_call(s0, s1, d0, d1, cnt, batch2,
               eW, eb2, W1, b12, W2, b22, W3, b32):
    full = lambda shape: pl.BlockSpec(shape, lambda i: (0, 0))
    return pl.pallas_call(
        _head_body,
        grid=(N // BN,),
        in_specs=[
            pl.BlockSpec((BN, NHID), lambda i: (i, 0)),
            pl.BlockSpec((BN, NHID), lambda i: (i, 0)),
            pl.BlockSpec((BN, 16), lambda i: (i, 0)),
            pl.BlockSpec((BN, 16), lambda i: (i, 0)),
            pl.BlockSpec((BN, 1), lambda i: (i, 0)),
            pl.BlockSpec((BN, 1), lambda i: (i, 0)),
            full((NHID, NHID)), full((1, NHID)),
            full((2 * NHID, NHID)), full((1, NHID)),
            full((NHID, 64)), full((1, 64)),
            full((64, 4)), full((1, 4)),
        ],
        out_specs=pl.BlockSpec((NG, 4), lambda i: (0, 0)),
        out_shape=jax.ShapeDtypeStruct((NG, 4), _F32),
        scratch_shapes=[
            pltpu.VMEM((NG, NHID), _F32),
            pltpu.VMEM((NG, NHID), _F32),
            pltpu.VMEM((NG, NHID), _F32),
            pltpu.VMEM((NG, NHID), _F32),
        ],
    )(s0, s1, d0, d1, cnt, batch2, eW, eb2, W1, b12, W2, b22, W3, b32)


def kernel(x, edge_index, pin_feature, batch, fake_pos, macro_index,
           v2e_node_W, v2e_node_b, v2e_pin_W, v2e_pin_b,
           e2v_W, e2v_b, att,
           mlp_W1, mlp_b1, mlp_W2, mlp_b2, mlp_W3, mlp_b3,
           mlp2_W1, mlp2_b1, mlp2_W2, mlp2_b2, mlp2_W3, mlp2_b3):
    src = edge_index[0]
    dst = edge_index[1]
    xf = jnp.concatenate([x, fake_pos, jnp.zeros((N, 1), _F32)], axis=1)

    h, ha, cnt = _node_call(
        xf, macro_index.reshape(1, NMACRO), v2e_node_W,
        v2e_node_b.reshape(1, NHID), att[:NHID].reshape(1, NHID))

    hsrc, had = _gather_kernel(
        h, ha.reshape(N), src.reshape(E // CH, CH), dst)

    e, wattr, exv = _edge_call(
        hsrc, pin_feature, had.reshape(E, 1),
        v2e_pin_W, v2e_pin_b.reshape(1, NHID),
        att[NHID:].reshape(1, NHID),
        mlp2_W1, mlp2_b1.reshape(1, 64),
        mlp2_W2, mlp2_b2.reshape(1, 32),
        mlp2_W3, mlp2_b3.reshape(1, 4))

    sparts, dparts = _scatter_kernel(
        wattr, exv, dst.reshape(E // CH, CH),
        jnp.zeros((ROWS_W, NHID), _F32), jnp.zeros((ROWS_W, 16), _F32))

    g = _head_call(
        sparts[:N], sparts[N:], dparts[:N], dparts[N:],
        cnt, batch.reshape(N, 1),
        e2v_W, e2v_b.reshape(1, NHID),
        mlp_W1, mlp_b1.reshape(1, NHID),
        mlp_W2, mlp_b2.reshape(1, 64),
        mlp_W3, mlp_b3.reshape(1, 4))

    return (g, e)


# speedup vs baseline: 3.8671x; 3.8671x over previous
import functools

import jax
import jax.numpy as jnp
from jax import lax
from jax.experimental import pallas as pl
from jax.experimental.pallas import tpu as pltpu
from jax.experimental.pallas import tpu_sc as plsc

N = 10000
E = 320000
NG = 16
NHID = 128
NMACRO = 512

NC = 2
NS = 16
NW = NC * NS
PER_W = E // NW
CH = 80
NCHUNK = PER_W // CH
NP = 10240

NROWS = NHID + 8
CHK = 2000
NCH2 = E // CHK
NEX = PER_W // CHK

BN = 1000
BE = 2560
BH = 1024

_F32 = jnp.float32


def _node_body(xf_ref, macro_ref, W_ref, b_ref, att1_ref,
               h_ref, ha_ref, cnt_ref):
    i = pl.program_id(0)
    rid = lax.broadcasted_iota(jnp.int32, (BN, NMACRO), 0) + i * BN
    eqf = (rid == macro_ref[...]).astype(_F32)
    ones_m = jnp.ones((1, NMACRO), _F32)
    dn = (((1,), (1,)), ((), ()))
    cnt = lax.dot_general(eqf, ones_m, dn, preferred_element_type=_F32)
    ismacro = (cnt > 0).astype(_F32)
    h = jnp.dot(xf_ref[...], W_ref[...], preferred_element_type=_F32)
    h = h + ismacro * W_ref[NHID - 1:NHID, :] + b_ref[...]
    h_ref[...] = h
    ha_ref[...] = lax.dot_general(h, att1_ref[...], dn,
                                  preferred_element_type=_F32)
    cnt_ref[...] = cnt


def _node_call(xf, macro2, W, b2, att1):
    return pl.pallas_call(
        _node_body,
        grid=(N // BN,),
        in_specs=[
            pl.BlockSpec((BN, NHID), lambda i: (i, 0)),
            pl.BlockSpec((1, NMACRO), lambda i: (0, 0)),
            pl.BlockSpec((NHID, NHID), lambda i: (0, 0)),
            pl.BlockSpec((1, NHID), lambda i: (0, 0)),
            pl.BlockSpec((1, NHID), lambda i: (0, 0)),
        ],
        out_specs=[
            pl.BlockSpec((BN, NHID), lambda i: (i, 0)),
            pl.BlockSpec((BN, 1), lambda i: (i, 0)),
            pl.BlockSpec((BN, 1), lambda i: (i, 0)),
        ],
        out_shape=[
            jax.ShapeDtypeStruct((N, NHID), _F32),
            jax.ShapeDtypeStruct((N, 1), _F32),
            jax.ShapeDtypeStruct((N, 1), _F32),
        ],
    )(xf, macro2, W, b2, att1)


_MESH = plsc.VectorSubcoreMesh(core_axis_name="c", subcore_axis_name="s",
                               num_cores=NC, num_subcores=NS)


@functools.partial(
    pl.kernel,
    out_type=(jax.ShapeDtypeStruct((E, NHID), _F32),
              jax.ShapeDtypeStruct((E,), _F32)),
    mesh=_MESH,
    scratch_types=[
        pltpu.VMEM((CH,), jnp.int32),
        pltpu.VMEM((PER_W,), jnp.int32),
        pltpu.VMEM((N,), _F32),
        pltpu.VMEM((PER_W,), _F32),
        pltpu.VMEM((CH, NHID), _F32),
        pltpu.SemaphoreType.DMA,
    ],
    compiler_params=pltpu.CompilerParams(needs_layout_passes=False),
)
def _gather_kernel(h_hbm, ha_hbm, src_hbm, dst_hbm, hsrc_hbm, had_hbm,
                   sidx_v, didx_v, ha_v, had_v, rbuf, sem):
    wid = lax.axis_index("s") * NC + lax.axis_index("c")
    base = wid * PER_W
    pltpu.sync_copy(dst_hbm.at[pl.ds(base, PER_W)], didx_v)
    pltpu.sync_copy(ha_hbm, ha_v)

    def gath16(k, carry):
        idx16 = didx_v[pl.ds(k * 16, 16)]
        had_v[pl.ds(k * 16, 16)] = plsc.load_gather(ha_v, [idx16])
        return carry
    lax.fori_loop(0, PER_W // 16, gath16, 0)
    pltpu.sync_copy(had_v, had_hbm.at[pl.ds(base, PER_W)])

    def chunk(j, carry):
        pltpu.sync_copy(src_hbm.at[pl.ds(base + j * CH, CH)], sidx_v)
        pltpu.async_copy(h_hbm.at[sidx_v], rbuf, sem).wait()
        pltpu.sync_copy(rbuf, hsrc_hbm.at[pl.ds(base + j * CH, CH)])
        return carry
    lax.fori_loop(0, NCHUNK, chunk, 0)


def _edge_body(hsrc_ref, pin_ref, had_ref, pW_ref, pb_ref, att2_ref,
               W1_ref, b1_ref, W2_ref, b2_ref, W3_ref, b3_ref,
               e_ref, wT_ref):
    dn = (((1,), (1,)), ((), ()))
    pinp = jnp.dot(pin_ref[...], pW_ref[...],
                   preferred_element_type=_F32) + pb_ref[...]
    ea = jnp.maximum(hsrc_ref[...] + pinp, 0.0)
    z = had_ref[...] + lax.dot_general(ea, att2_ref[...], dn,
                                       preferred_element_type=_F32)
    z = jnp.where(z >= 0, z, 0.1 * z)
    ex = jnp.exp(z)
    wfull = jnp.concatenate(
        [ex * ea, ex, jnp.zeros((BE, NROWS - NHID - 1), _F32)], axis=1)
    ri = lax.broadcasted_iota(jnp.int32, (NROWS, NROWS), 0)
    ci = lax.broadcasted_iota(jnp.int32, (NROWS, NROWS), 1)
    eye = (ri == ci).astype(_F32)
    wT_ref[...] = lax.dot_general(eye, wfull, dn,
                                  preferred_element_type=_F32)
    v = jnp.dot(ea, W1_ref[...], preferred_element_type=_F32) + b1_ref[...]
    v = jnp.where(v >= 0, v, 0.1 * v)
    v = jnp.dot(v, W2_ref[...], preferred_element_type=_F32) + b2_ref[...]
    v = jnp.where(v >= 0, v, 0.1 * v)
    e_ref[...] = jnp.dot(v, W3_ref[...], preferred_element_type=_F32) + b3_ref[...]


def _edge_call(hsrc, pin, had2, pW, pb2, att2, W1, b12, W2, b22, W3, b32):
    full = lambda shape: pl.BlockSpec(shape, lambda i: (0, 0))
    return pl.pallas_call(
        _edge_body,
        grid=(E // BE,),
        in_specs=[
            pl.BlockSpec((BE, NHID), lambda i: (i, 0)),
            pl.BlockSpec((BE, 4), lambda i: (i, 0)),
            pl.BlockSpec((BE, 1), lambda i: (i, 0)),
            full((4, NHID)), full((1, NHID)), full((1, NHID)),
            full((NHID, 64)), full((1, 64)),
            full((64, 32)), full((1, 32)),
            full((32, 4)), full((1, 4)),
        ],
        out_specs=[
            pl.BlockSpec((BE, 4), lambda i: (i, 0)),
            pl.BlockSpec((NROWS, BE), lambda i: (0, i)),
        ],
        out_shape=[
            jax.ShapeDtypeStruct((E, 4), _F32),
            jax.ShapeDtypeStruct((NROWS, E), _F32),
        ],
    )(hsrc, pin, had2, pW, pb2, att2, W1, b12, W2, b22, W3, b32)


@functools.partial(
    pl.kernel,
    out_type=(jax.ShapeDtypeStruct((NHID * NP,), _F32),
              jax.ShapeDtypeStruct((NW * NP,), _F32)),
    mesh=_MESH,
    scratch_types=[
        pltpu.VMEM((NP,), _F32),
        pltpu.VMEM((NP,), _F32),
        pltpu.VMEM((NP,), _F32),
        pltpu.VMEM((NP,), _F32),
        pltpu.VMEM((NP,), _F32),
        pltpu.VMEM((CHK,), jnp.int32),
        pltpu.VMEM((CHK,), _F32),
        pltpu.VMEM((CHK,), _F32),
        pltpu.VMEM((CHK,), _F32),
        pltpu.VMEM((CHK,), _F32),
    ],
    compiler_params=pltpu.CompilerParams(needs_layout_passes=False),
)
def _acc_kernel(wT_hbm, dst_hbm, sacc_hbm, tparts_hbm,
                acc0, acc1, acc2, acc3, tacc, didx_v, db0, db1, db2, db3):
    wid = lax.axis_index("s") * NC + lax.axis_index("c")
    r0 = wid * 4

    zero16 = jnp.zeros((16,), _F32)

    def zloop(i, carry):
        sl = pl.ds(i * 16, 16)
        acc0[sl] = zero16
        acc1[sl] = zero16
        acc2[sl] = zero16
        acc3[sl] = zero16
        tacc[sl] = zero16
        return carry
    lax.fori_loop(0, NP // 16, zloop, 0)

    def chunk(j, carry):
        e0 = j * CHK
        pltpu.sync_copy(dst_hbm.at[pl.ds(e0, CHK)], didx_v)
        pltpu.sync_copy(wT_hbm.at[pl.ds((r0 + 0) * E + e0, CHK)], db0)
        pltpu.sync_copy(wT_hbm.at[pl.ds((r0 + 1) * E + e0, CHK)], db1)
        pltpu.sync_copy(wT_hbm.at[pl.ds((r0 + 2) * E + e0, CHK)], db2)
        pltpu.sync_copy(wT_hbm.at[pl.ds((r0 + 3) * E + e0, CHK)], db3)

        def inner(k, c2):
            sl = pl.ds(k * 16, 16)
            idx16 = didx_v[sl]
            plsc.addupdate_scatter(acc0, [idx16], db0[sl])
            plsc.addupdate_scatter(acc1, [idx16], db1[sl])
            plsc.addupdate_scatter(acc2, [idx16], db2[sl])
            plsc.addupdate_scatter(acc3, [idx16], db3[sl])
            return c2
        lax.fori_loop(0, CHK // 16, inner, 0)
        return carry
    lax.fori_loop(0, NCH2, chunk, 0)

    def exchunk(j, carry):
        e0 = wid * PER_W + j * CHK
        pltpu.sync_copy(dst_hbm.at[pl.ds(e0, CHK)], didx_v)
        pltpu.sync_copy(wT_hbm.at[pl.ds(NHID * E + e0, CHK)], db0)

        def inner(k, c2):
            sl = pl.ds(k * 16, 16)
            plsc.addupdate_scatter(tacc, [didx_v[sl]], db0[sl])
            return c2
        lax.fori_loop(0, CHK // 16, inner, 0)
        return carry
    lax.fori_loop(0, NEX, exchunk, 0)

    pltpu.sync_copy(acc0, sacc_hbm.at[pl.ds((r0 + 0) * NP, NP)])
    pltpu.sync_copy(acc1, sacc_hbm.at[pl.ds((r0 + 1) * NP, NP)])
    pltpu.sync_copy(acc2, sacc_hbm.at[pl.ds((r0 + 2) * NP, NP)])
    pltpu.sync_copy(acc3, sacc_hbm.at[pl.ds((r0 + 3) * NP, NP)])
    pltpu.sync_copy(tacc, tparts_hbm.at[pl.ds(wid * NP, NP)])


def _head_body(sT_ref, tp_ref, cnt_ref, batch_ref,
               eW_ref, eb_ref, W1_ref, b1_ref, W2_ref, b2_ref, W3_ref, b3_ref,
               g_ref, pm_acc, pf_acc, cm_acc, cf_acc):
    i = pl.program_id(0)

    @pl.when(i == 0)
    def _init():
        pm_acc[...] = jnp.zeros_like(pm_acc)
        pf_acc[...] = jnp.zeros_like(pf_acc)
        cm_acc[...] = jnp.zeros_like(cm_acc)
        cf_acc[...] = jnp.zeros_like(cf_acc)

    dn0 = (((0,), (0,)), ((), ()))
    t = lax.dot_general(tp_ref[...], jnp.ones((1, NW), _F32),
                        (((0,), (1,)), ((), ())),
                        preferred_element_type=_F32)
    xo = (lax.dot_general(sT_ref[...], eW_ref[...], dn0,
                          preferred_element_type=_F32)
          + t * eb_ref[...]) / (t + 1e-16)
    gi = lax.broadcasted_iota(jnp.int32, (BH, NG), 1)
    oh = (batch_ref[...] == gi).astype(_F32)
    ohm = oh * cnt_ref[...]
    ones = jnp.ones((BH, NHID), _F32)
    pf_acc[...] += lax.dot_general(oh, xo, dn0, preferred_element_type=_F32)
    pm_acc[...] += lax.dot_general(ohm, xo, dn0, preferred_element_type=_F32)
    cf_acc[...] += lax.dot_general(oh, ones, dn0, preferred_element_type=_F32)
    cm_acc[...] += lax.dot_general(ohm, ones, dn0, preferred_element_type=_F32)

    @pl.when(i == NP // BH - 1)
    def _final():
        x1a = pm_acc[...] / jnp.maximum(cm_acc[...], 1.0)
        x1b = pf_acc[...] / jnp.maximum(cf_acc[...], 1.0)
        v = (jnp.dot(x1a, W1_ref[...][:NHID], preferred_element_type=_F32)
             + jnp.dot(x1b, W1_ref[...][NHID:], preferred_element_type=_F32)
             + b1_ref[...])
        v = jnp.where(v >= 0, v, 0.1 * v)
        v = jnp.dot(v, W2_ref[...], preferred_element_type=_F32) + b2_ref[...]
        v = jnp.where(v >= 0, v, 0.1 * v)
        g_ref[...] = jnp.dot(v, W3_ref[...],
                             preferred_element_type=_F32) + b3_ref[...]


def _head_call(sT, tp, cnt, batch2,
               eW, eb2, W1, b12, W2, b22, W3, b32):
    full = lambda shape: pl.BlockSpec(shape, lambda i: (0, 0))
    return pl.pallas_call(
        _head_body,
        grid=(NP // BH,),
        in_specs=[
            pl.BlockSpec((NHID, BH), lambda i: (0, i)),
            pl.BlockSpec((NW, BH), lambda i: (0, i)),
            pl.BlockSpec((BH, 1), lambda i: (i, 0)),
            pl.BlockSpec((BH, 1), lambda i: (i, 0)),
            full((NHID, NHID)), full((1, NHID)),
            full((2 * NHID, NHID)), full((1, NHID)),
            full((NHID, 64)), full((1, 64)),
            full((64, 4)), full((1, 4)),
        ],
        out_specs=pl.BlockSpec((NG, 4), lambda i: (0, 0)),
        out_shape=jax.ShapeDtypeStruct((NG, 4), _F32),
        scratch_shapes=[
            pltpu.VMEM((NG, NHID), _F32),
            pltpu.VMEM((NG, NHID), _F32),
            pltpu.VMEM((NG, NHID), _F32),
            pltpu.VMEM((NG, NHID), _F32),
        ],
    )(sT, tp, cnt, batch2, eW, eb2, W1, b12, W2, b22, W3, b32)


def kernel(x, edge_index, pin_feature, batch, fake_pos, macro_index,
           v2e_node_W, v2e_node_b, v2e_pin_W, v2e_pin_b,
           e2v_W, e2v_b, att,
           mlp_W1, mlp_b1, mlp_W2, mlp_b2, mlp_W3, mlp_b3,
           mlp2_W1, mlp2_b1, mlp2_W2, mlp2_b2, mlp2_W3, mlp2_b3):
    src = edge_index[0]
    dst = edge_index[1]
    xf = jnp.concatenate([x, fake_pos, jnp.zeros((N, 1), _F32)], axis=1)

    h, ha, cnt = _node_call(
        xf, macro_index.reshape(1, NMACRO), v2e_node_W,
        v2e_node_b.reshape(1, NHID), att[:NHID].reshape(1, NHID))

    hsrc, had = _gather_kernel(h, ha.reshape(N), src, dst)

    e, wT = _edge_call(
        hsrc, pin_feature, had.reshape(E, 1),
        v2e_pin_W, v2e_pin_b.reshape(1, NHID),
        att[NHID:].reshape(1, NHID),
        mlp2_W1, mlp2_b1.reshape(1, 64),
        mlp2_W2, mlp2_b2.reshape(1, 32),
        mlp2_W3, mlp2_b3.reshape(1, 4))

    sacc_f, tparts_f = _acc_kernel(wT.reshape(NROWS * E), dst)
    sT = sacc_f.reshape(NHID, NP)
    tp = tparts_f.reshape(NW, NP)
    cnt_p = jnp.concatenate([cnt, jnp.zeros((NP - N, 1), _F32)], axis=0)
    batch_p = jnp.concatenate(
        [batch, jnp.full((NP - N,), -1, jnp.int32)]).reshape(NP, 1)

    g = _head_call(
        sT, tp, cnt_p, batch_p,
        e2v_W, e2v_b.reshape(1, NHID),
        mlp_W1, mlp_b1.reshape(1, NHID),
        mlp_W2, mlp_b2.reshape(1, 64),
        mlp_W3, mlp_b3.reshape(1, 4))

    return (g, e)


# speedup vs baseline: 5.3794x; 1.3911x over previous
import functools

import jax
import jax.numpy as jnp
from jax import lax
from jax.experimental import pallas as pl
from jax.experimental.pallas import tpu as pltpu
from jax.experimental.pallas import tpu_sc as plsc

N = 10000
E = 320000
NG = 16
NHID = 128
NMACRO = 512

NC = 2
NS = 16
NW = NC * NS
PER_W = E // NW
CH = 80
NCHUNK = PER_W // CH
NP = 10240

NROWS = NHID + 8
CHK = 2000
NCH2 = E // CHK
NEX = PER_W // CHK

BN = 1000
BE = 2560
BH = 1024

_F32 = jnp.float32


def _node_body(xf_ref, macro_ref, W_ref, b_ref, att1_ref,
               h_ref, ha_ref, cnt_ref):
    i = pl.program_id(0)
    rid = lax.broadcasted_iota(jnp.int32, (BN, NMACRO), 0) + i * BN
    eqf = (rid == macro_ref[...]).astype(_F32)
    ones_m = jnp.ones((1, NMACRO), _F32)
    dn = (((1,), (1,)), ((), ()))
    cnt = lax.dot_general(eqf, ones_m, dn, preferred_element_type=_F32)
    ismacro = (cnt > 0).astype(_F32)
    h = jnp.dot(xf_ref[...], W_ref[...], preferred_element_type=_F32)
    h = h + ismacro * W_ref[NHID - 1:NHID, :] + b_ref[...]
    h_ref[...] = h
    ha_ref[...] = lax.dot_general(h, att1_ref[...], dn,
                                  preferred_element_type=_F32)
    cnt_ref[...] = cnt


def _node_call(xf, macro2, W, b2, att1):
    return pl.pallas_call(
        _node_body,
        grid=(N // BN,),
        in_specs=[
            pl.BlockSpec((BN, NHID), lambda i: (i, 0)),
            pl.BlockSpec((1, NMACRO), lambda i: (0, 0)),
            pl.BlockSpec((NHID, NHID), lambda i: (0, 0)),
            pl.BlockSpec((1, NHID), lambda i: (0, 0)),
            pl.BlockSpec((1, NHID), lambda i: (0, 0)),
        ],
        out_specs=[
            pl.BlockSpec((BN, NHID), lambda i: (i, 0)),
            pl.BlockSpec((BN, 1), lambda i: (i, 0)),
            pl.BlockSpec((BN, 1), lambda i: (i, 0)),
        ],
        out_shape=[
            jax.ShapeDtypeStruct((N, NHID), _F32),
            jax.ShapeDtypeStruct((N, 1), _F32),
            jax.ShapeDtypeStruct((N, 1), _F32),
        ],
    )(xf, macro2, W, b2, att1)


_MESH = plsc.VectorSubcoreMesh(core_axis_name="c", subcore_axis_name="s",
                               num_cores=NC, num_subcores=NS)


@functools.partial(
    pl.kernel,
    out_type=(jax.ShapeDtypeStruct((E, NHID), _F32),
              jax.ShapeDtypeStruct((E,), _F32)),
    mesh=_MESH,
    scratch_types=[
        pltpu.VMEM((CH,), jnp.int32),
        pltpu.VMEM((PER_W,), jnp.int32),
        pltpu.VMEM((N,), _F32),
        pltpu.VMEM((PER_W,), _F32),
        pltpu.VMEM((CH, NHID), _F32),
        pltpu.SemaphoreType.DMA,
    ],
    compiler_params=pltpu.CompilerParams(needs_layout_passes=False),
)
def _gather_kernel(h_hbm, ha_hbm, src_hbm, dst_hbm, hsrc_hbm, had_hbm,
                   sidx_v, didx_v, ha_v, had_v, rbuf, sem):
    wid = lax.axis_index("s") * NC + lax.axis_index("c")
    base = wid * PER_W
    pltpu.sync_copy(dst_hbm.at[pl.ds(base, PER_W)], didx_v)
    pltpu.sync_copy(ha_hbm, ha_v)

    def gath16(k, carry):
        idx16 = didx_v[pl.ds(k * 16, 16)]
        had_v[pl.ds(k * 16, 16)] = plsc.load_gather(ha_v, [idx16])
        return carry
    lax.fori_loop(0, PER_W // 16, gath16, 0)
    pltpu.sync_copy(had_v, had_hbm.at[pl.ds(base, PER_W)])

    def chunk(j, carry):
        pltpu.sync_copy(src_hbm.at[pl.ds(base + j * CH, CH)], sidx_v)
        pltpu.async_copy(h_hbm.at[sidx_v], rbuf, sem).wait()
        pltpu.sync_copy(rbuf, hsrc_hbm.at[pl.ds(base + j * CH, CH)])
        return carry
    lax.fori_loop(0, NCHUNK, chunk, 0)


def _edge_body(hsrc_ref, pin_ref, had_ref, pW_ref, pb_ref, att2_ref,
               W1_ref, b1_ref, W2_ref, b2_ref, W3_ref, b3_ref,
               e_ref, wT_ref):
    dn = (((1,), (1,)), ((), ()))
    pinp = jnp.dot(pin_ref[...], pW_ref[...],
                   preferred_element_type=_F32) + pb_ref[...]
    ea = jnp.maximum(hsrc_ref[...] + pinp, 0.0)
    z = had_ref[...] + lax.dot_general(ea, att2_ref[...], dn,
                                       preferred_element_type=_F32)
    z = jnp.where(z >= 0, z, 0.1 * z)
    ex = jnp.exp(z)
    wfull = jnp.concatenate(
        [ex * ea, ex, jnp.zeros((BE, NROWS - NHID - 1), _F32)], axis=1)
    ri = lax.broadcasted_iota(jnp.int32, (NROWS, NROWS), 0)
    ci = lax.broadcasted_iota(jnp.int32, (NROWS, NROWS), 1)
    eye = (ri == ci).astype(_F32)
    wT_ref[...] = lax.dot_general(eye, wfull, dn,
                                  preferred_element_type=_F32)
    v = jnp.dot(ea, W1_ref[...], preferred_element_type=_F32) + b1_ref[...]
    v = jnp.where(v >= 0, v, 0.1 * v)
    v = jnp.dot(v, W2_ref[...], preferred_element_type=_F32) + b2_ref[...]
    v = jnp.where(v >= 0, v, 0.1 * v)
    e_ref[...] = jnp.dot(v, W3_ref[...], preferred_element_type=_F32) + b3_ref[...]


def _edge_call(hsrc, pin, had2, pW, pb2, att2, W1, b12, W2, b22, W3, b32):
    full = lambda shape: pl.BlockSpec(shape, lambda i: (0, 0))
    return pl.pallas_call(
        _edge_body,
        grid=(E // BE,),
        in_specs=[
            pl.BlockSpec((BE, NHID), lambda i: (i, 0)),
            pl.BlockSpec((BE, 4), lambda i: (i, 0)),
            pl.BlockSpec((BE, 1), lambda i: (i, 0)),
            full((4, NHID)), full((1, NHID)), full((1, NHID)),
            full((NHID, 64)), full((1, 64)),
            full((64, 32)), full((1, 32)),
            full((32, 4)), full((1, 4)),
        ],
        out_specs=[
            pl.BlockSpec((BE, 4), lambda i: (i, 0)),
            pl.BlockSpec((NROWS, BE), lambda i: (0, i)),
        ],
        out_shape=[
            jax.ShapeDtypeStruct((E, 4), _F32),
            jax.ShapeDtypeStruct((NROWS, E), _F32),
        ],
    )(hsrc, pin, had2, pW, pb2, att2, W1, b12, W2, b22, W3, b32)


@functools.partial(
    pl.kernel,
    out_type=(jax.ShapeDtypeStruct((NHID * NP,), _F32),
              jax.ShapeDtypeStruct((NW * NP,), _F32)),
    mesh=_MESH,
    scratch_types=[
        pltpu.VMEM((NP,), _F32),
        pltpu.VMEM((NP,), _F32),
        pltpu.VMEM((NP,), _F32),
        pltpu.VMEM((NP,), _F32),
        pltpu.VMEM((NP,), _F32),
        pltpu.VMEM((CHK,), jnp.int32),
        pltpu.VMEM((CHK,), _F32),
        pltpu.VMEM((CHK,), _F32),
        pltpu.VMEM((CHK,), _F32),
        pltpu.VMEM((CHK,), _F32),
        pltpu.VMEM((CHK,), jnp.int32),
        pltpu.VMEM((CHK,), _F32),
        pltpu.VMEM((CHK,), _F32),
        pltpu.VMEM((CHK,), _F32),
        pltpu.VMEM((CHK,), _F32),
        pltpu.SemaphoreType.DMA,
        pltpu.SemaphoreType.DMA,
    ],
    compiler_params=pltpu.CompilerParams(needs_layout_passes=False),
)
def _acc_kernel(wT_hbm, dst_hbm, sacc_hbm, tparts_hbm,
                acc0, acc1, acc2, acc3, tacc,
                didxA, da0, da1, da2, da3,
                didxB, dbb0, dbb1, dbb2, dbb3,
                sem0, sem1):
    wid = lax.axis_index("s") * NC + lax.axis_index("c")
    r0 = wid * 4
    accs = (acc0, acc1, acc2, acc3)
    sets = ((didxA, (da0, da1, da2, da3), sem0),
            (didxB, (dbb0, dbb1, dbb2, dbb3), sem1))

    zero16 = jnp.zeros((16,), _F32)

    def zloop(i, carry):
        sl = pl.ds(i * 16, 16)
        acc0[sl] = zero16
        acc1[sl] = zero16
        acc2[sl] = zero16
        acc3[sl] = zero16
        tacc[sl] = zero16
        return carry
    lax.fori_loop(0, NP // 16, zloop, 0)

    def issue(j, s):
        didx, dbs, sem = sets[s]
        e0 = j * CHK
        pltpu.async_copy(dst_hbm.at[pl.ds(e0, CHK)], didx, sem)
        for r in range(4):
            pltpu.async_copy(wT_hbm.at[pl.ds((r0 + r) * E + e0, CHK)],
                             dbs[r], sem)

    def drain(s):
        didx, dbs, sem = sets[s]
        pltpu.make_async_copy(dst_hbm.at[pl.ds(0, CHK)], didx, sem).wait()
        for r in range(4):
            pltpu.make_async_copy(wT_hbm.at[pl.ds(0, CHK)], dbs[r],
                                  sem).wait()

    def compute(s):
        didx, dbs, _ = sets[s]

        def inner(k, c2):
            for u in range(5):
                sl = pl.ds((k * 5 + u) * 16, 16)
                idx16 = didx[sl]
                for r in range(4):
                    plsc.addupdate_scatter(accs[r], [idx16], dbs[r][sl])
            return c2
        lax.fori_loop(0, CHK // 80, inner, 0)

    issue(0, 0)

    def pair(jj, carry):
        j0 = 2 * jj
        issue(j0 + 1, 1)
        drain(0)
        compute(0)

        @pl.when(jj < NCH2 // 2 - 1)
        def _next():
            issue(j0 + 2, 0)

        drain(1)
        compute(1)
        return carry
    lax.fori_loop(0, NCH2 // 2, pair, 0)

    def exchunk(j, carry):
        e0 = wid * PER_W + j * CHK
        pltpu.sync_copy(dst_hbm.at[pl.ds(e0, CHK)], didxA)
        pltpu.sync_copy(wT_hbm.at[pl.ds(NHID * E + e0, CHK)], da0)

        def inner(k, c2):
            sl = pl.ds(k * 16, 16)
            plsc.addupdate_scatter(tacc, [didxA[sl]], da0[sl])
            return c2
        lax.fori_loop(0, CHK // 16, inner, 0)
        return carry
    lax.fori_loop(0, NEX, exchunk, 0)

    pltpu.sync_copy(acc0, sacc_hbm.at[pl.ds((r0 + 0) * NP, NP)])
    pltpu.sync_copy(acc1, sacc_hbm.at[pl.ds((r0 + 1) * NP, NP)])
    pltpu.sync_copy(acc2, sacc_hbm.at[pl.ds((r0 + 2) * NP, NP)])
    pltpu.sync_copy(acc3, sacc_hbm.at[pl.ds((r0 + 3) * NP, NP)])
    pltpu.sync_copy(tacc, tparts_hbm.at[pl.ds(wid * NP, NP)])


def _head_body(sT_ref, tp_ref, cnt_ref, batch_ref,
               eW_ref, eb_ref, W1_ref, b1_ref, W2_ref, b2_ref, W3_ref, b3_ref,
               g_ref, pm_acc, pf_acc, cm_acc, cf_acc):
    i = pl.program_id(0)

    @pl.when(i == 0)
    def _init():
        pm_acc[...] = jnp.zeros_like(pm_acc)
        pf_acc[...] = jnp.zeros_like(pf_acc)
        cm_acc[...] = jnp.zeros_like(cm_acc)
        cf_acc[...] = jnp.zeros_like(cf_acc)

    dn0 = (((0,), (0,)), ((), ()))
    t = lax.dot_general(tp_ref[...], jnp.ones((1, NW), _F32),
                        (((0,), (1,)), ((), ())),
                        preferred_element_type=_F32)
    xo = (lax.dot_general(sT_ref[...], eW_ref[...], dn0,
                          preferred_element_type=_F32)
          + t * eb_ref[...]) / (t + 1e-16)
    gi = lax.broadcasted_iota(jnp.int32, (BH, NG), 1)
    oh = (batch_ref[...] == gi).astype(_F32)
    ohm = oh * cnt_ref[...]
    ones = jnp.ones((BH, NHID), _F32)
    pf_acc[...] += lax.dot_general(oh, xo, dn0, preferred_element_type=_F32)
    pm_acc[...] += lax.dot_general(ohm, xo, dn0, preferred_element_type=_F32)
    cf_acc[...] += lax.dot_general(oh, ones, dn0, preferred_element_type=_F32)
    cm_acc[...] += lax.dot_general(ohm, ones, dn0, preferred_element_type=_F32)

    @pl.when(i == NP // BH - 1)
    def _final():
        x1a = pm_acc[...] / jnp.maximum(cm_acc[...], 1.0)
        x1b = pf_acc[...] / jnp.maximum(cf_acc[...], 1.0)
        v = (jnp.dot(x1a, W1_ref[...][:NHID], preferred_element_type=_F32)
             + jnp.dot(x1b, W1_ref[...][NHID:], preferred_element_type=_F32)
             + b1_ref[...])
        v = jnp.where(v >= 0, v, 0.1 * v)
        v = jnp.dot(v, W2_ref[...], preferred_element_type=_F32) + b2_ref[...]
        v = jnp.where(v >= 0, v, 0.1 * v)
        g_ref[...] = jnp.dot(v, W3_ref[...],
                             preferred_element_type=_F32) + b3_ref[...]


def _head_call(sT, tp, cnt, batch2,
               eW, eb2, W1, b12, W2, b22, W3, b32):
    full = lambda shape: pl.BlockSpec(shape, lambda i: (0, 0))
    return pl.pallas_call(
        _head_body,
        grid=(NP // BH,),
        in_specs=[
            pl.BlockSpec((NHID, BH), lambda i: (0, i)),
            pl.BlockSpec((NW, BH), lambda i: (0, i)),
            pl.BlockSpec((BH, 1), lambda i: (i, 0)),
            pl.BlockSpec((BH, 1), lambda i: (i, 0)),
            full((NHID, NHID)), full((1, NHID)),
            full((2 * NHID, NHID)), full((1, NHID)),
            full((NHID, 64)), full((1, 64)),
            full((64, 4)), full((1, 4)),
        ],
        out_specs=pl.BlockSpec((NG, 4), lambda i: (0, 0)),
        out_shape=jax.ShapeDtypeStruct((NG, 4), _F32),
        scratch_shapes=[
            pltpu.VMEM((NG, NHID), _F32),
            pltpu.VMEM((NG, NHID), _F32),
            pltpu.VMEM((NG, NHID), _F32),
            pltpu.VMEM((NG, NHID), _F32),
        ],
    )(sT, tp, cnt, batch2, eW, eb2, W1, b12, W2, b22, W3, b32)


def kernel(x, edge_index, pin_feature, batch, fake_pos, macro_index,
           v2e_node_W, v2e_node_b, v2e_pin_W, v2e_pin_b,
           e2v_W, e2v_b, att,
           mlp_W1, mlp_b1, mlp_W2, mlp_b2, mlp_W3, mlp_b3,
           mlp2_W1, mlp2_b1, mlp2_W2, mlp2_b2, mlp2_W3, mlp2_b3):
    src = edge_index[0]
    dst = edge_index[1]
    xf = jnp.concatenate([x, fake_pos, jnp.zeros((N, 1), _F32)], axis=1)

    h, ha, cnt = _node_call(
        xf, macro_index.reshape(1, NMACRO), v2e_node_W,
        v2e_node_b.reshape(1, NHID), att[:NHID].reshape(1, NHID))

    hsrc, had = _gather_kernel(h, ha.reshape(N), src, dst)

    e, wT = _edge_call(
        hsrc, pin_feature, had.reshape(E, 1),
        v2e_pin_W, v2e_pin_b.reshape(1, NHID),
        att[NHID:].reshape(1, NHID),
        mlp2_W1, mlp2_b1.reshape(1, 64),
        mlp2_W2, mlp2_b2.reshape(1, 32),
        mlp2_W3, mlp2_b3.reshape(1, 4))

    sacc_f, tparts_f = _acc_kernel(wT.reshape(NROWS * E), dst)
    sT = sacc_f.reshape(NHID, NP)
    tp = tparts_f.reshape(NW, NP)
    cnt_p = jnp.concatenate([cnt, jnp.zeros((NP - N, 1), _F32)], axis=0)
    batch_p = jnp.concatenate(
        [batch, jnp.full((NP - N,), -1, jnp.int32)]).reshape(NP, 1)

    g = _head_call(
        sT, tp, cnt_p, batch_p,
        e2v_W, e2v_b.reshape(1, NHID),
        mlp_W1, mlp_b1.reshape(1, NHID),
        mlp_W2, mlp_b2.reshape(1, 64),
        mlp_W3, mlp_b3.reshape(1, 4))

    return (g, e)


# speedup vs baseline: 5.8894x; 1.0948x over previous
import functools

import jax
import jax.numpy as jnp
from jax import lax
from jax.experimental import pallas as pl
from jax.experimental.pallas import tpu as pltpu
from jax.experimental.pallas import tpu_sc as plsc

N = 10000
E = 320000
NG = 16
NHID = 128
NMACRO = 512

NC = 2
NS = 16
NW = NC * NS
PER_W = E // NW
CH = 80
NCHUNK = PER_W // CH
NP = 10240

NROWS = NHID + 8
CHK = 2000
NCH2 = E // CHK
NEX = PER_W // CHK

BN = 1000
BE = 2560
BH = 1024

_F32 = jnp.float32


def _node_body(xf_ref, macro_ref, W_ref, b_ref, att1_ref,
               h_ref, ha_ref, cnt_ref):
    i = pl.program_id(0)
    rid = lax.broadcasted_iota(jnp.int32, (BN, NMACRO), 0) + i * BN
    eqf = (rid == macro_ref[...]).astype(_F32)
    ones_m = jnp.ones((1, NMACRO), _F32)
    dn = (((1,), (1,)), ((), ()))
    cnt = lax.dot_general(eqf, ones_m, dn, preferred_element_type=_F32)
    ismacro = (cnt > 0).astype(_F32)
    h = jnp.dot(xf_ref[...], W_ref[...], preferred_element_type=_F32)
    h = h + ismacro * W_ref[NHID - 1:NHID, :] + b_ref[...]
    h_ref[...] = h
    ha_ref[...] = lax.dot_general(h, att1_ref[...], dn,
                                  preferred_element_type=_F32)
    cnt_ref[...] = cnt


def _node_call(xf, macro2, W, b2, att1):
    return pl.pallas_call(
        _node_body,
        grid=(N // BN,),
        in_specs=[
            pl.BlockSpec((BN, NHID), lambda i: (i, 0)),
            pl.BlockSpec((1, NMACRO), lambda i: (0, 0)),
            pl.BlockSpec((NHID, NHID), lambda i: (0, 0)),
            pl.BlockSpec((1, NHID), lambda i: (0, 0)),
            pl.BlockSpec((1, NHID), lambda i: (0, 0)),
        ],
        out_specs=[
            pl.BlockSpec((BN, NHID), lambda i: (i, 0)),
            pl.BlockSpec((BN, 1), lambda i: (i, 0)),
            pl.BlockSpec((BN, 1), lambda i: (i, 0)),
        ],
        out_shape=[
            jax.ShapeDtypeStruct((N, NHID), _F32),
            jax.ShapeDtypeStruct((N, 1), _F32),
            jax.ShapeDtypeStruct((N, 1), _F32),
        ],
    )(xf, macro2, W, b2, att1)


_MESH = plsc.VectorSubcoreMesh(core_axis_name="c", subcore_axis_name="s",
                               num_cores=NC, num_subcores=NS)


@functools.partial(
    pl.kernel,
    out_type=(jax.ShapeDtypeStruct((E, NHID), _F32),
              jax.ShapeDtypeStruct((E,), _F32)),
    mesh=_MESH,
    scratch_types=[
        pltpu.VMEM((PER_W,), jnp.int32),
        pltpu.VMEM((PER_W,), jnp.int32),
        pltpu.VMEM((N,), _F32),
        pltpu.VMEM((PER_W,), _F32),
        pltpu.VMEM((CH, NHID), _F32),
        pltpu.VMEM((CH, NHID), _F32),
        pltpu.SemaphoreType.DMA,
        pltpu.SemaphoreType.DMA,
        pltpu.SemaphoreType.DMA,
        pltpu.SemaphoreType.DMA,
        pltpu.SemaphoreType.DMA,
    ],
    compiler_params=pltpu.CompilerParams(needs_layout_passes=False),
)
def _gather_kernel(h_hbm, ha_hbm, src_hbm, dst_hbm, hsrc_hbm, had_hbm,
                   sidx_v, didx_v, ha_v, had_v, rbufA, rbufB,
                   semL, semGA, semWA, semGB, semWB):
    wid = lax.axis_index("s") * NC + lax.axis_index("c")
    base = wid * PER_W
    pltpu.async_copy(src_hbm.at[pl.ds(base, PER_W)], sidx_v, semL)
    pltpu.async_copy(dst_hbm.at[pl.ds(base, PER_W)], didx_v, semL)
    pltpu.async_copy(ha_hbm, ha_v, semL)
    pltpu.make_async_copy(src_hbm.at[pl.ds(0, PER_W)], sidx_v, semL).wait()
    pltpu.make_async_copy(dst_hbm.at[pl.ds(0, PER_W)], didx_v, semL).wait()
    pltpu.make_async_copy(ha_hbm, ha_v, semL).wait()

    def issue_g(j, buf, sem):
        pltpu.async_copy(h_hbm.at[sidx_v.at[pl.ds(j * CH, CH)]], buf, sem)

    def wait_g(buf, sem):
        pltpu.make_async_copy(h_hbm.at[sidx_v.at[pl.ds(0, CH)]], buf,
                              sem).wait()

    def issue_w(j, buf, sem):
        pltpu.async_copy(buf, hsrc_hbm.at[pl.ds(base + j * CH, CH)], sem)

    def wait_w(buf, sem):
        pltpu.make_async_copy(buf, hsrc_hbm.at[pl.ds(0, CH)], sem).wait()

    issue_g(0, rbufA, semGA)

    def gath16(k, carry):
        idx16 = didx_v[pl.ds(k * 16, 16)]
        had_v[pl.ds(k * 16, 16)] = plsc.load_gather(ha_v, [idx16])
        return carry
    lax.fori_loop(0, PER_W // 16, gath16, 0)
    pltpu.sync_copy(had_v, had_hbm.at[pl.ds(base, PER_W)])

    def pair(jj, carry):
        j0 = 2 * jj
        issue_g(j0 + 1, rbufB, semGB)
        wait_g(rbufA, semGA)
        issue_w(j0, rbufA, semWA)
        wait_g(rbufB, semGB)
        issue_w(j0 + 1, rbufB, semWB)
        wait_w(rbufA, semWA)
        issue_g(j0 + 2, rbufA, semGA)
        wait_w(rbufB, semWB)
        return carry
    lax.fori_loop(0, NCHUNK // 2, pair, 0)

    wait_g(rbufA, semGA)
    issue_w(NCHUNK - 1, rbufA, semWA)
    wait_w(rbufA, semWA)


def _edge_body(hsrc_ref, pin_ref, had_ref, pW_ref, pb_ref, att2_ref,
               W1_ref, b1_ref, W2_ref, b2_ref, W3_ref, b3_ref,
               e_ref, wT_ref):
    dn = (((1,), (1,)), ((), ()))
    pinp = jnp.dot(pin_ref[...], pW_ref[...],
                   preferred_element_type=_F32) + pb_ref[...]
    ea = jnp.maximum(hsrc_ref[...] + pinp, 0.0)
    z = had_ref[...] + lax.dot_general(ea, att2_ref[...], dn,
                                       preferred_element_type=_F32)
    z = jnp.where(z >= 0, z, 0.1 * z)
    ex = jnp.exp(z)
    wfull = jnp.concatenate(
        [ex * ea, ex, jnp.zeros((BE, NROWS - NHID - 1), _F32)], axis=1)
    ri = lax.broadcasted_iota(jnp.int32, (NROWS, NROWS), 0)
    ci = lax.broadcasted_iota(jnp.int32, (NROWS, NROWS), 1)
    eye = (ri == ci).astype(_F32)
    wT_ref[...] = lax.dot_general(eye, wfull, dn,
                                  preferred_element_type=_F32)
    v = jnp.dot(ea, W1_ref[...], preferred_element_type=_F32) + b1_ref[...]
    v = jnp.where(v >= 0, v, 0.1 * v)
    v = jnp.dot(v, W2_ref[...], preferred_element_type=_F32) + b2_ref[...]
    v = jnp.where(v >= 0, v, 0.1 * v)
    e_ref[...] = jnp.dot(v, W3_ref[...], preferred_element_type=_F32) + b3_ref[...]


def _edge_call(hsrc, pin, had2, pW, pb2, att2, W1, b12, W2, b22, W3, b32):
    full = lambda shape: pl.BlockSpec(shape, lambda i: (0, 0))
    return pl.pallas_call(
        _edge_body,
        grid=(E // BE,),
        in_specs=[
            pl.BlockSpec((BE, NHID), lambda i: (i, 0)),
            pl.BlockSpec((BE, 4), lambda i: (i, 0)),
            pl.BlockSpec((BE, 1), lambda i: (i, 0)),
            full((4, NHID)), full((1, NHID)), full((1, NHID)),
            full((NHID, 64)), full((1, 64)),
            full((64, 32)), full((1, 32)),
            full((32, 4)), full((1, 4)),
        ],
        out_specs=[
            pl.BlockSpec((BE, 4), lambda i: (i, 0)),
            pl.BlockSpec((NROWS, BE), lambda i: (0, i)),
        ],
        out_shape=[
            jax.ShapeDtypeStruct((E, 4), _F32),
            jax.ShapeDtypeStruct((NROWS, E), _F32),
        ],
    )(hsrc, pin, had2, pW, pb2, att2, W1, b12, W2, b22, W3, b32)


@functools.partial(
    pl.kernel,
    out_type=(jax.ShapeDtypeStruct((NHID * NP,), _F32),
              jax.ShapeDtypeStruct((NW * NP,), _F32)),
    mesh=_MESH,
    scratch_types=[
        pltpu.VMEM((NP,), _F32),
        pltpu.VMEM((NP,), _F32),
        pltpu.VMEM((NP,), _F32),
        pltpu.VMEM((NP,), _F32),
        pltpu.VMEM((NP,), _F32),
        pltpu.VMEM((CHK,), jnp.int32),
        pltpu.VMEM((CHK,), _F32),
        pltpu.VMEM((CHK,), _F32),
        pltpu.VMEM((CHK,), _F32),
        pltpu.VMEM((CHK,), _F32),
        pltpu.VMEM((CHK,), jnp.int32),
        pltpu.VMEM((CHK,), _F32),
        pltpu.VMEM((CHK,), _F32),
        pltpu.VMEM((CHK,), _F32),
        pltpu.VMEM((CHK,), _F32),
        pltpu.SemaphoreType.DMA,
        pltpu.SemaphoreType.DMA,
    ],
    compiler_params=pltpu.CompilerParams(needs_layout_passes=False),
)
def _acc_kernel(wT_hbm, dst_hbm, sacc_hbm, tparts_hbm,
                acc0, acc1, acc2, acc3, tacc,
                didxA, da0, da1, da2, da3,
                didxB, dbb0, dbb1, dbb2, dbb3,
                sem0, sem1):
    wid = lax.axis_index("s") * NC + lax.axis_index("c")
    r0 = wid * 4
    accs = (acc0, acc1, acc2, acc3)
    sets = ((didxA, (da0, da1, da2, da3), sem0),
            (didxB, (dbb0, dbb1, dbb2, dbb3), sem1))

    zero16 = jnp.zeros((16,), _F32)

    def zloop(i, carry):
        sl = pl.ds(i * 16, 16)
        acc0[sl] = zero16
        acc1[sl] = zero16
        acc2[sl] = zero16
        acc3[sl] = zero16
        tacc[sl] = zero16
        return carry
    lax.fori_loop(0, NP // 16, zloop, 0)

    def issue(j, s):
        didx, dbs, sem = sets[s]
        e0 = j * CHK
        pltpu.async_copy(dst_hbm.at[pl.ds(e0, CHK)], didx, sem)
        for r in range(4):
            pltpu.async_copy(wT_hbm.at[pl.ds((r0 + r) * E + e0, CHK)],
                             dbs[r], sem)

    def drain(s):
        didx, dbs, sem = sets[s]
        pltpu.make_async_copy(dst_hbm.at[pl.ds(0, CHK)], didx, sem).wait()
        for r in range(4):
            pltpu.make_async_copy(wT_hbm.at[pl.ds(0, CHK)], dbs[r],
                                  sem).wait()

    def compute(s):
        didx, dbs, _ = sets[s]

        def inner(k, c2):
            for u in range(5):
                sl = pl.ds((k * 5 + u) * 16, 16)
                idx16 = didx[sl]
                for r in range(4):
                    plsc.addupdate_scatter(accs[r], [idx16], dbs[r][sl])
            return c2
        lax.fori_loop(0, CHK // 80, inner, 0)

    issue(0, 0)

    def pair(jj, carry):
        j0 = 2 * jj
        issue(j0 + 1, 1)
        drain(0)
        compute(0)

        @pl.when(jj < NCH2 // 2 - 1)
        def _next():
            issue(j0 + 2, 0)

        drain(1)
        compute(1)
        return carry
    lax.fori_loop(0, NCH2 // 2, pair, 0)

    def exchunk(j, carry):
        e0 = wid * PER_W + j * CHK
        pltpu.sync_copy(dst_hbm.at[pl.ds(e0, CHK)], didxA)
        pltpu.sync_copy(wT_hbm.at[pl.ds(NHID * E + e0, CHK)], da0)

        def inner(k, c2):
            sl = pl.ds(k * 16, 16)
            plsc.addupdate_scatter(tacc, [didxA[sl]], da0[sl])
            return c2
        lax.fori_loop(0, CHK // 16, inner, 0)
        return carry
    lax.fori_loop(0, NEX, exchunk, 0)

    pltpu.sync_copy(acc0, sacc_hbm.at[pl.ds((r0 + 0) * NP, NP)])
    pltpu.sync_copy(acc1, sacc_hbm.at[pl.ds((r0 + 1) * NP, NP)])
    pltpu.sync_copy(acc2, sacc_hbm.at[pl.ds((r0 + 2) * NP, NP)])
    pltpu.sync_copy(acc3, sacc_hbm.at[pl.ds((r0 + 3) * NP, NP)])
    pltpu.sync_copy(tacc, tparts_hbm.at[pl.ds(wid * NP, NP)])


def _head_body(sT_ref, tp_ref, cnt_ref, batch_ref,
               eW_ref, eb_ref, W1_ref, b1_ref, W2_ref, b2_ref, W3_ref, b3_ref,
               g_ref, pm_acc, pf_acc, cm_acc, cf_acc):
    i = pl.program_id(0)

    @pl.when(i == 0)
    def _init():
        pm_acc[...] = jnp.zeros_like(pm_acc)
        pf_acc[...] = jnp.zeros_like(pf_acc)
        cm_acc[...] = jnp.zeros_like(cm_acc)
        cf_acc[...] = jnp.zeros_like(cf_acc)

    dn0 = (((0,), (0,)), ((), ()))
    t = lax.dot_general(tp_ref[...], jnp.ones((1, NW), _F32),
                        (((0,), (1,)), ((), ())),
                        preferred_element_type=_F32)
    xo = (lax.dot_general(sT_ref[...], eW_ref[...], dn0,
                          preferred_element_type=_F32)
          + t * eb_ref[...]) / (t + 1e-16)
    gi = lax.broadcasted_iota(jnp.int32, (BH, NG), 1)
    oh = (batch_ref[...] == gi).astype(_F32)
    ohm = oh * cnt_ref[...]
    ones = jnp.ones((BH, NHID), _F32)
    pf_acc[...] += lax.dot_general(oh, xo, dn0, preferred_element_type=_F32)
    pm_acc[...] += lax.dot_general(ohm, xo, dn0, preferred_element_type=_F32)
    cf_acc[...] += lax.dot_general(oh, ones, dn0, preferred_element_type=_F32)
    cm_acc[...] += lax.dot_general(ohm, ones, dn0, preferred_element_type=_F32)

    @pl.when(i == NP // BH - 1)
    def _final():
        x1a = pm_acc[...] / jnp.maximum(cm_acc[...], 1.0)
        x1b = pf_acc[...] / jnp.maximum(cf_acc[...], 1.0)
        v = (jnp.dot(x1a, W1_ref[...][:NHID], preferred_element_type=_F32)
             + jnp.dot(x1b, W1_ref[...][NHID:], preferred_element_type=_F32)
             + b1_ref[...])
        v = jnp.where(v >= 0, v, 0.1 * v)
        v = jnp.dot(v, W2_ref[...], preferred_element_type=_F32) + b2_ref[...]
        v = jnp.where(v >= 0, v, 0.1 * v)
        g_ref[...] = jnp.dot(v, W3_ref[...],
                             preferred_element_type=_F32) + b3_ref[...]


def _head_call(sT, tp, cnt, batch2,
               eW, eb2, W1, b12, W2, b22, W3, b32):
    full = lambda shape: pl.BlockSpec(shape, lambda i: (0, 0))
    return pl.pallas_call(
        _head_body,
        grid=(NP // BH,),
        in_specs=[
            pl.BlockSpec((NHID, BH), lambda i: (0, i)),
            pl.BlockSpec((NW, BH), lambda i: (0, i)),
            pl.BlockSpec((BH, 1), lambda i: (i, 0)),
            pl.BlockSpec((BH, 1), lambda i: (i, 0)),
            full((NHID, NHID)), full((1, NHID)),
            full((2 * NHID, NHID)), full((1, NHID)),
            full((NHID, 64)), full((1, 64)),
            full((64, 4)), full((1, 4)),
        ],
        out_specs=pl.BlockSpec((NG, 4), lambda i: (0, 0)),
        out_shape=jax.ShapeDtypeStruct((NG, 4), _F32),
        scratch_shapes=[
            pltpu.VMEM((NG, NHID), _F32),
            pltpu.VMEM((NG, NHID), _F32),
            pltpu.VMEM((NG, NHID), _F32),
            pltpu.VMEM((NG, NHID), _F32),
        ],
    )(sT, tp, cnt, batch2, eW, eb2, W1, b12, W2, b22, W3, b32)


def kernel(x, edge_index, pin_feature, batch, fake_pos, macro_index,
           v2e_node_W, v2e_node_b, v2e_pin_W, v2e_pin_b,
           e2v_W, e2v_b, att,
           mlp_W1, mlp_b1, mlp_W2, mlp_b2, mlp_W3, mlp_b3,
           mlp2_W1, mlp2_b1, mlp2_W2, mlp2_b2, mlp2_W3, mlp2_b3):
    src = edge_index[0]
    dst = edge_index[1]
    xf = jnp.concatenate([x, fake_pos, jnp.zeros((N, 1), _F32)], axis=1)

    h, ha, cnt = _node_call(
        xf, macro_index.reshape(1, NMACRO), v2e_node_W,
        v2e_node_b.reshape(1, NHID), att[:NHID].reshape(1, NHID))

    hsrc, had = _gather_kernel(h, ha.reshape(N), src, dst)

    e, wT = _edge_call(
        hsrc, pin_feature, had.reshape(E, 1),
        v2e_pin_W, v2e_pin_b.reshape(1, NHID),
        att[NHID:].reshape(1, NHID),
        mlp2_W1, mlp2_b1.reshape(1, 64),
        mlp2_W2, mlp2_b2.reshape(1, 32),
        mlp2_W3, mlp2_b3.reshape(1, 4))

    sacc_f, tparts_f = _acc_kernel(wT.reshape(NROWS * E), dst)
    sT = sacc_f.reshape(NHID, NP)
    tp = tparts_f.reshape(NW, NP)
    cnt_p = jnp.concatenate([cnt, jnp.zeros((NP - N, 1), _F32)], axis=0)
    batch_p = jnp.concatenate(
        [batch, jnp.full((NP - N,), -1, jnp.int32)]).reshape(NP, 1)

    g = _head_call(
        sT, tp, cnt_p, batch_p,
        e2v_W, e2v_b.reshape(1, NHID),
        mlp_W1, mlp_b1.reshape(1, NHID),
        mlp_W2, mlp_b2.reshape(1, 64),
        mlp_W3, mlp_b3.reshape(1, 4))

    return (g, e)
